# Initial kernel scaffold; baseline (speedup 1.0000x reference)
import jax
import jax.numpy as jnp
from jax.experimental import pallas as pl


def kernel(x, edge_index, lin_l_w, lin_l_b, lin_r_w, norm1_w, norm1_b, norm1_ms, norm2_w, norm2_b, norm2_ms):
    raise NotImplementedError("write your pallas kernel here")


# speedup vs baseline: 4.4476x; 4.4476x over previous
import functools

import jax
import jax.numpy as jnp
from jax import lax
from jax.experimental import pallas as pl
from jax.experimental.pallas import tpu as pltpu
from jax.experimental.pallas import tpu_sc as plsc

N = 10000
E = 320000
DIM = 128
EPS = 1e-5

NC = 2
NS = 16
CH = 128
CPT = 160
NCHUNK = NS * CPT
E_PAD = NCHUNK * CH
HF = DIM // 2
D_AUG = HF + 16
NPAD = N + 112
ROWS_PT = NPAD // NS


def _sc_aggregate(x2, src2d, dst2d):
    mesh = plsc.VectorSubcoreMesh(core_axis_name="c", subcore_axis_name="s")

    @functools.partial(
        pl.kernel,
        out_type=jax.ShapeDtypeStruct((NC, NPAD, D_AUG), jnp.float32),
        mesh=mesh,
        scratch_types=[
            pltpu.VMEM((CPT, CH), jnp.int32),
            pltpu.VMEM((CPT, CH), jnp.int32),
            pltpu.VMEM((2, CH, D_AUG), jnp.float32),
            pltpu.VMEM_SHARED((NPAD, D_AUG), jnp.float32),
            pltpu.SemaphoreType.DMA,
            pltpu.SemaphoreType.DMA,
        ],
        compiler_params=pltpu.CompilerParams(use_tc_tiling_on_sc=False),
    )
    def body(x_hbm, src_hbm, dst_hbm, agg_out, src_v, dst_v, rows_v, agg_sp,
             sem0, sem1):
        c = lax.axis_index("c")
        s = lax.axis_index("s")
        sems = (sem0, sem1)
        zero16 = jnp.zeros((16,), jnp.float32)
        table = x_hbm.at[c]

        @pl.loop(0, CH)
        def _(i):
            for j in range(D_AUG // 16):
                rows_v[0, i, pl.ds(j * 16, 16)] = zero16

        row0 = s * ROWS_PT
        for k in range(4):
            pltpu.sync_copy(rows_v.at[0, pl.ds(0, CH)],
                            agg_sp.at[pl.ds(row0 + k * CH, CH)])
        rem = ROWS_PT - 4 * CH
        pltpu.sync_copy(rows_v.at[0, pl.ds(0, rem)],
                        agg_sp.at[pl.ds(row0 + 4 * CH, rem)])
        plsc.subcore_barrier()

        base = s * CPT
        pltpu.sync_copy(src_hbm.at[pl.ds(base, CPT)], src_v)
        pltpu.sync_copy(dst_hbm.at[pl.ds(base, CPT)], dst_v)

        def start(j, b):
            pltpu.async_copy(table.at[src_v.at[j]], rows_v.at[b], sems[b])

        def wait(j, b):
            pltpu.make_async_copy(table.at[src_v.at[j]], rows_v.at[b],
                                  sems[b]).wait()

        def scatter(j, b):
            pltpu.sync_copy(rows_v.at[b], agg_sp.at[dst_v.at[j]], add=True)

        start(0, 0)
        start(1, 1)

        @pl.loop(0, CPT // 2 - 1)
        def _(i):
            for b in range(2):
                j = 2 * i + b
                wait(j, b)
                start(j + 2, b)
                scatter(j, b)

        for b in range(2):
            j = CPT - 2 + b
            wait(j, b)
            scatter(j, b)

        plsc.subcore_barrier()
        pltpu.sync_copy(agg_sp.at[pl.ds(row0, ROWS_PT)],
                        agg_out.at[c, pl.ds(row0, ROWS_PT)])

    return body(x2, src2d, dst2d)


def _tc_body(x_ref, agg_ref, wl_ref, bl_ref, wr_ref, w2_ref, b2_ref, ms2_ref,
             out_ref):
    x = x_ref[...]
    cnt = agg_ref[0, :N, HF:HF + 1]
    inv = 1.0 / jnp.maximum(cnt, 1.0)
    lo = agg_ref[0, :N, :HF] * inv
    hi = agg_ref[1, :N, :HF] * inv
    wl = wl_ref[...]
    h = (x
         + jnp.dot(lo, wl[:HF, :], preferred_element_type=jnp.float32)
         + jnp.dot(hi, wl[HF:, :], preferred_element_type=jnp.float32)
         + bl_ref[...]
         + jnp.dot(x, wr_ref[...], preferred_element_type=jnp.float32))
    h = jnp.maximum(h, 0.0)
    n = jnp.float32(N)
    s1 = jnp.sum(h, axis=0, keepdims=True)
    s2 = jnp.sum(h * h, axis=0, keepdims=True)
    mean = s1 / n
    ms = ms2_ref[...]
    var = s2 / n + (ms * ms - 2.0 * ms) * mean * mean
    hc = h - ms * mean
    out_ref[...] = w2_ref[...] * (hc * lax.rsqrt(var + EPS)) + b2_ref[...]


def _tc_finish(x, agg_parts, wl, bl, wr, w2, b2, ms2):
    return pl.pallas_call(
        _tc_body,
        out_shape=jax.ShapeDtypeStruct((N, DIM), jnp.float32),
    )(x, agg_parts, wl, bl, wr, w2, b2, ms2)


def kernel(x, edge_index, lin_l_w, lin_l_b, lin_r_w, norm1_w, norm1_b,
           norm1_ms, norm2_w, norm2_b, norm2_ms):
    ones = jnp.ones((N, 16), jnp.float32)
    x2 = jnp.stack([
        jnp.concatenate([x[:, :HF], ones], axis=1),
        jnp.concatenate([x[:, HF:], ones], axis=1),
    ])
    src = edge_index[0]
    dst = edge_index[1]
    pad = E_PAD - E
    src_p = jnp.concatenate([src, jnp.zeros((pad,), jnp.int32)]
                            ).reshape(NCHUNK, CH)
    dst_p = jnp.concatenate([dst, jnp.full((pad,), N, jnp.int32)]
                            ).reshape(NCHUNK, CH)
    agg_parts = _sc_aggregate(x2, src_p, dst_p)
    return _tc_finish(
        x, agg_parts,
        lin_l_w.T, lin_l_b.reshape(1, DIM), lin_r_w.T,
        norm2_w.reshape(1, DIM), norm2_b.reshape(1, DIM),
        norm2_ms.reshape(1, DIM),
    )
